# baseline (speedup 1.0000x reference)
import functools

import jax
import jax.numpy as jnp
from jax import lax
from jax.experimental import pallas as pl
from jax.experimental.pallas import tpu as pltpu
from jax.experimental.pallas import tpu_sc as plsc

N = 50000
M = 16
ATOM = 64
NBR = 41
F = 2 * ATOM
E = N * M
EPS = 1e-5


_TN0 = 2048
_G0 = (N + _TN0 - 1) // _TN0


def _k0_body(atomt_ref, wst_ref, wnt_ref, b_ref, p_ref, s_ref):
    at = atomt_ref[...]
    dn = (((0,), (0,)), ((), ()))
    p_ref[...] = lax.dot_general(at, wnt_ref[...], dn,
                                 preferred_element_type=jnp.float32)
    s_ref[...] = lax.dot_general(at, wst_ref[...], dn,
                                 preferred_element_type=jnp.float32) + b_ref[...]


def _k0(atomt, wst, wnt, b2d):
    return pl.pallas_call(
        _k0_body,
        grid=(_G0,),
        in_specs=[
            pl.BlockSpec((ATOM, _TN0), lambda i: (0, i)),
            pl.BlockSpec((ATOM, F), lambda i: (0, 0)),
            pl.BlockSpec((ATOM, F), lambda i: (0, 0)),
            pl.BlockSpec((1, F), lambda i: (0, 0)),
        ],
        out_specs=[
            pl.BlockSpec((_TN0, F), lambda i: (i, 0)),
            pl.BlockSpec((_TN0, F), lambda i: (i, 0)),
        ],
        out_shape=[
            jax.ShapeDtypeStruct((N, F), jnp.float32),
            jax.ShapeDtypeStruct((N, F), jnp.float32),
        ],
    )(atomt, wst, wnt, b2d)


_NC, _NS = 2, 16
_NW = _NC * _NS
_NG = 4
_MG = M // _NG
_WS = _NW // _MG
_CH = 200
_CPS = N // _CH


def _sc_gather(p_tbl, idx_t, g):
    mesh = plsc.VectorSubcoreMesh(core_axis_name="c", subcore_axis_name="s",
                                  num_cores=_NC)

    @functools.partial(
        pl.kernel,
        mesh=mesh,
        out_type=jax.ShapeDtypeStruct((_MG * N, F), jnp.float32),
        scratch_types=[
            pltpu.VMEM((N,), jnp.int32),
            pltpu.VMEM((_CH, F), jnp.float32),
            pltpu.VMEM((_CH, F), jnp.float32),
            pltpu.SemaphoreType.DMA,
            pltpu.SemaphoreType.DMA,
            pltpu.SemaphoreType.DMA,
        ],
    )
    def gather_k(tbl_hbm, idx_hbm, out_hbm, idx_v, rows0, rows1, sem_g,
                 sem_w0, sem_w1):
        wid = lax.axis_index("s") * _NC + lax.axis_index("c")
        ml = wid // _WS
        wl = lax.rem(wid, _WS)
        base_ch = 31 * wl + jnp.maximum(wl - 6, 0)
        nch = jnp.where(wl >= 6, 32, 31)
        base = base_ch * _CH
        pltpu.sync_copy(idx_hbm.at[g * _MG + ml], idx_v)
        obase = ml * N + base

        def body(ch, _):
            b = lax.rem(ch, 2)

            def chunk(rows_v, sem_w):
                @pl.when(ch >= 2)
                def _():
                    pltpu.make_async_copy(
                        rows_v, out_hbm.at[pl.ds(obase, _CH)], sem_w).wait()

                off = ch * _CH
                pltpu.async_copy(
                    tbl_hbm.at[idx_v.at[pl.ds(base + off, _CH)]], rows_v,
                    sem_g).wait()
                pltpu.async_copy(rows_v, out_hbm.at[pl.ds(obase + off, _CH)],
                                 sem_w)

            @pl.when(b == 0)
            def _():
                chunk(rows0, sem_w0)

            @pl.when(b == 1)
            def _():
                chunk(rows1, sem_w1)

            return 0

        lax.fori_loop(0, nch, body, 0)
        pltpu.make_async_copy(rows0, out_hbm.at[pl.ds(obase, _CH)],
                              sem_w0).wait()
        pltpu.make_async_copy(rows1, out_hbm.at[pl.ds(obase, _CH)],
                              sem_w1).wait()

    return gather_k(p_tbl, idx_t)


_TN1 = 1024
_G1 = (N + _TN1 - 1) // _TN1


def _p1_body(nbrt_ref, g_ref, s_ref, wet_ref, x_ref, s1_ref, s2_ref):
    i = pl.program_id(0)
    s = s_ref[...]
    rem = N - i * _TN1
    rmask = lax.broadcasted_iota(jnp.int32, (_TN1, F), 0) < rem
    dn = (((0,), (0,)), ((), ()))
    acc1 = jnp.zeros((1, F), jnp.float32)
    acc2 = jnp.zeros((1, F), jnp.float32)
    for m in range(_MG):
        ey = lax.dot_general(nbrt_ref[:, 0, m, :], wet_ref[...], dn,
                             preferred_element_type=jnp.float32)
        xm = g_ref[m] + ey + s
        x_ref[m] = xm.astype(jnp.bfloat16)
        xv = jnp.where(rmask, xm, 0.0)
        acc1 += jnp.sum(xv, axis=0).reshape(1, F)
        acc2 += jnp.sum(xv * xv, axis=0).reshape(1, F)

    @pl.when(i == 0)
    def _():
        s1_ref[...] = jnp.zeros_like(s1_ref)
        s2_ref[...] = jnp.zeros_like(s2_ref)

    s1_ref[...] += acc1
    s2_ref[...] += acc2


def _pass1(nbrt, g3, s, wet, g):
    return pl.pallas_call(
        _p1_body,
        grid=(_G1,),
        in_specs=[
            pl.BlockSpec((NBR, 1, _MG, _TN1), lambda i: (0, g, 0, i)),
            pl.BlockSpec((_MG, _TN1, F), lambda i: (0, i, 0)),
            pl.BlockSpec((_TN1, F), lambda i: (i, 0)),
            pl.BlockSpec((NBR, F), lambda i: (0, 0)),
        ],
        out_specs=[
            pl.BlockSpec((_MG, _TN1, F), lambda i: (0, i, 0)),
            pl.BlockSpec((1, F), lambda i: (0, 0)),
            pl.BlockSpec((1, F), lambda i: (0, 0)),
        ],
        out_shape=[
            jax.ShapeDtypeStruct((_MG, N, F), jnp.bfloat16),
            jax.ShapeDtypeStruct((1, F), jnp.float32),
            jax.ShapeDtypeStruct((1, F), jnp.float32),
        ],
    )(nbrt, g3, s, wet)


def _p2_body(x0_ref, x1_ref, x2_ref, x3_ref, s1_ref, s2_ref, g1_ref, b1_ref,
             ns_ref, t1_ref, t2_ref):
    i = pl.program_id(0)
    inv = 1.0 / float(E)
    mu = jnp.sum(s1_ref[...], axis=0, keepdims=True) * inv
    var = jnp.sum(s2_ref[...], axis=0, keepdims=True) * inv - mu * mu
    a1 = g1_ref[...] * lax.rsqrt(var + EPS)
    c1 = b1_ref[...] - mu * a1
    ns = jnp.zeros((_TN1, ATOM), jnp.float32)
    for x_ref in (x0_ref, x1_ref, x2_ref, x3_ref):
        z = a1[None] * x_ref[...].astype(jnp.float32) + c1[None]
        filt = jax.nn.sigmoid(z[:, :, :ATOM])
        core = jax.nn.softplus(z[:, :, ATOM:])
        ns = ns + jnp.sum(filt * core, axis=0)
    ns_ref[...] = ns

    @pl.when(i == 0)
    def _():
        t1_ref[...] = jnp.zeros_like(t1_ref)
        t2_ref[...] = jnp.zeros_like(t2_ref)

    rem = N - i * _TN1
    rmask = lax.broadcasted_iota(jnp.int32, (_TN1, ATOM), 0) < rem
    nsv = jnp.where(rmask, ns, 0.0)
    t1_ref[...] += jnp.sum(nsv, axis=0).reshape(1, ATOM)
    t2_ref[...] += jnp.sum(nsv * nsv, axis=0).reshape(1, ATOM)


def _pass2(xs, s1, s2, g1, b1):
    return pl.pallas_call(
        _p2_body,
        grid=(_G1,),
        in_specs=[
            pl.BlockSpec((_MG, _TN1, F), lambda i: (0, i, 0)),
            pl.BlockSpec((_MG, _TN1, F), lambda i: (0, i, 0)),
            pl.BlockSpec((_MG, _TN1, F), lambda i: (0, i, 0)),
            pl.BlockSpec((_MG, _TN1, F), lambda i: (0, i, 0)),
            pl.BlockSpec((_NG, F), lambda i: (0, 0)),
            pl.BlockSpec((_NG, F), lambda i: (0, 0)),
            pl.BlockSpec((1, F), lambda i: (0, 0)),
            pl.BlockSpec((1, F), lambda i: (0, 0)),
        ],
        out_specs=[
            pl.BlockSpec((_TN1, ATOM), lambda i: (i, 0)),
            pl.BlockSpec((1, ATOM), lambda i: (0, 0)),
            pl.BlockSpec((1, ATOM), lambda i: (0, 0)),
        ],
        out_shape=[
            jax.ShapeDtypeStruct((N, ATOM), jnp.float32),
            jax.ShapeDtypeStruct((1, ATOM), jnp.float32),
            jax.ShapeDtypeStruct((1, ATOM), jnp.float32),
        ],
    )(*xs, s1, s2, g1, b1)


_TN3 = 2000
_G3 = N // _TN3


def _p3_body(atom_ref, ns_ref, t1_ref, t2_ref, g2_ref, b2_ref, out_ref):
    inv = 1.0 / float(N)
    mu2 = t1_ref[...] * inv
    var2 = t2_ref[...] * inv - mu2 * mu2
    a2c = g2_ref[...] * lax.rsqrt(var2 + EPS)
    c2c = b2_ref[...] - mu2 * a2c
    out_ref[...] = jax.nn.softplus(atom_ref[...] + a2c * ns_ref[...] + c2c)


def _pass3(atom, ns, t1, t2, g2, b2):
    return pl.pallas_call(
        _p3_body,
        grid=(_G3,),
        in_specs=[
            pl.BlockSpec((_TN3, ATOM), lambda i: (i, 0)),
            pl.BlockSpec((_TN3, ATOM), lambda i: (i, 0)),
            pl.BlockSpec((1, ATOM), lambda i: (0, 0)),
            pl.BlockSpec((1, ATOM), lambda i: (0, 0)),
            pl.BlockSpec((1, ATOM), lambda i: (0, 0)),
            pl.BlockSpec((1, ATOM), lambda i: (0, 0)),
        ],
        out_specs=pl.BlockSpec((_TN3, ATOM), lambda i: (i, 0)),
        out_shape=jax.ShapeDtypeStruct((N, ATOM), jnp.float32),
    )(atom, ns, t1, t2, g2, b2)


def kernel(atom_in_fea, nbr_fea, nbr_fea_idx, W, b, bn1_g, bn1_b, bn2_g,
           bn2_b):
    atomt = atom_in_fea.T
    nbrt = jnp.transpose(nbr_fea, (2, 1, 0))
    idx_t = nbr_fea_idx.T.astype(jnp.int32)
    wst = W[:, :ATOM].T
    wnt = W[:, ATOM:2 * ATOM].T
    wet = W[:, 2 * ATOM:].T
    b2d = b.reshape(1, F)

    p_tbl, s_tbl = _k0(atomt, wst, wnt, b2d)
    xs, s1s, s2s = [], [], []
    nbrt4 = nbrt.reshape(NBR, _NG, _MG, N)
    for g in range(_NG):
        gg = _sc_gather(p_tbl, idx_t, g)
        g3 = gg.reshape(_MG, N, F)
        x, s1, s2 = _pass1(nbrt4, g3, s_tbl, wet, g)
        xs.append(x)
        s1s.append(s1)
        s2s.append(s2)
    ns, t1, t2 = _pass2(xs, jnp.concatenate(s1s), jnp.concatenate(s2s),
                        bn1_g.reshape(1, F), bn1_b.reshape(1, F))
    return _pass3(atom_in_fea, ns, t1, t2, bn2_g.reshape(1, ATOM),
                  bn2_b.reshape(1, ATOM))


# speedup vs baseline: 1.3004x; 1.3004x over previous
import functools

import jax
import jax.numpy as jnp
from jax import lax
from jax.experimental import pallas as pl
from jax.experimental.pallas import tpu as pltpu
from jax.experimental.pallas import tpu_sc as plsc

N = 50000
M = 16
ATOM = 64
NBR = 41
F = 2 * ATOM
E = N * M
EPS = 1e-5


_TN0 = 2048
_G0 = (N + _TN0 - 1) // _TN0


def _k0_body(atomt_ref, wst_ref, wnt_ref, b_ref, p_ref, s_ref):
    at = atomt_ref[...]
    dn = (((0,), (0,)), ((), ()))
    p_ref[...] = lax.dot_general(at, wnt_ref[...], dn,
                                 preferred_element_type=jnp.float32)
    s_ref[...] = lax.dot_general(at, wst_ref[...], dn,
                                 preferred_element_type=jnp.float32) + b_ref[...]


def _k0(atomt, wst, wnt, b2d):
    return pl.pallas_call(
        _k0_body,
        grid=(_G0,),
        in_specs=[
            pl.BlockSpec((ATOM, _TN0), lambda i: (0, i)),
            pl.BlockSpec((ATOM, F), lambda i: (0, 0)),
            pl.BlockSpec((ATOM, F), lambda i: (0, 0)),
            pl.BlockSpec((1, F), lambda i: (0, 0)),
        ],
        out_specs=[
            pl.BlockSpec((_TN0, F), lambda i: (i, 0)),
            pl.BlockSpec((_TN0, F), lambda i: (i, 0)),
        ],
        out_shape=[
            jax.ShapeDtypeStruct((N, F), jnp.float32),
            jax.ShapeDtypeStruct((N, F), jnp.float32),
        ],
    )(atomt, wst, wnt, b2d)


_NC, _NS = 2, 16
_NW = _NC * _NS
_HN = N // 2
_CH = 200
_NCH = _HN // _CH


def _sc_gather(p_tbl, idx_t):
    mesh = plsc.VectorSubcoreMesh(core_axis_name="c", subcore_axis_name="s",
                                  num_cores=_NC)

    @functools.partial(
        pl.kernel,
        mesh=mesh,
        out_type=jax.ShapeDtypeStruct((E, F), jnp.float32),
        scratch_types=[
            pltpu.VMEM((N,), jnp.int32),
            pltpu.VMEM((_CH, F), jnp.float32),
            pltpu.VMEM((_CH, F), jnp.float32),
            pltpu.SemaphoreType.DMA,
            pltpu.SemaphoreType.DMA,
            pltpu.SemaphoreType.DMA,
        ],
    )
    def gather_k(tbl_hbm, idx_hbm, out_hbm, idx_v, rows0, rows1, sem_g,
                 sem_w0, sem_w1):
        wid = lax.axis_index("s") * _NC + lax.axis_index("c")
        m = wid // 2
        base = lax.rem(wid, 2) * _HN
        pltpu.sync_copy(idx_hbm.at[m], idx_v)
        obase = m * N + base

        def body(ch, _):
            b = lax.rem(ch, 2)

            def chunk(rows_v, sem_w):
                @pl.when(ch >= 2)
                def _():
                    pltpu.make_async_copy(
                        rows_v, out_hbm.at[pl.ds(obase, _CH)], sem_w).wait()

                off = ch * _CH
                pltpu.async_copy(
                    tbl_hbm.at[idx_v.at[pl.ds(base + off, _CH)]], rows_v,
                    sem_g).wait()
                pltpu.async_copy(rows_v, out_hbm.at[pl.ds(obase + off, _CH)],
                                 sem_w)

            @pl.when(b == 0)
            def _():
                chunk(rows0, sem_w0)

            @pl.when(b == 1)
            def _():
                chunk(rows1, sem_w1)

            return 0

        lax.fori_loop(0, _NCH, body, 0)
        pltpu.make_async_copy(rows0, out_hbm.at[pl.ds(obase, _CH)],
                              sem_w0).wait()
        pltpu.make_async_copy(rows1, out_hbm.at[pl.ds(obase, _CH)],
                              sem_w1).wait()

    return gather_k(p_tbl, idx_t)


_TN1 = 1024
_G1 = (N + _TN1 - 1) // _TN1


def _p1_body(nbrt_ref, g_ref, s_ref, wet_ref, x_ref, s1_ref, s2_ref):
    i = pl.program_id(0)
    s = s_ref[...]
    rem = N - i * _TN1
    rmask = lax.broadcasted_iota(jnp.int32, (_TN1, F), 0) < rem
    dn = (((0,), (0,)), ((), ()))
    acc1 = jnp.zeros((1, F), jnp.float32)
    acc2 = jnp.zeros((1, F), jnp.float32)
    for m in range(M):
        ey = lax.dot_general(nbrt_ref[:, m, :], wet_ref[...], dn,
                             preferred_element_type=jnp.float32)
        xm = g_ref[m] + ey + s
        x_ref[m] = xm.astype(jnp.bfloat16)
        xv = jnp.where(rmask, xm, 0.0)
        acc1 += jnp.sum(xv, axis=0).reshape(1, F)
        acc2 += jnp.sum(xv * xv, axis=0).reshape(1, F)

    @pl.when(i == 0)
    def _():
        s1_ref[...] = jnp.zeros_like(s1_ref)
        s2_ref[...] = jnp.zeros_like(s2_ref)

    s1_ref[...] += acc1
    s2_ref[...] += acc2


def _pass1(nbrt, g3, s, wet):
    return pl.pallas_call(
        _p1_body,
        grid=(_G1,),
        in_specs=[
            pl.BlockSpec((NBR, M, _TN1), lambda i: (0, 0, i)),
            pl.BlockSpec((M, _TN1, F), lambda i: (0, i, 0)),
            pl.BlockSpec((_TN1, F), lambda i: (i, 0)),
            pl.BlockSpec((NBR, F), lambda i: (0, 0)),
        ],
        out_specs=[
            pl.BlockSpec((M, _TN1, F), lambda i: (0, i, 0)),
            pl.BlockSpec((1, F), lambda i: (0, 0)),
            pl.BlockSpec((1, F), lambda i: (0, 0)),
        ],
        out_shape=[
            jax.ShapeDtypeStruct((M, N, F), jnp.bfloat16),
            jax.ShapeDtypeStruct((1, F), jnp.float32),
            jax.ShapeDtypeStruct((1, F), jnp.float32),
        ],
    )(nbrt, g3, s, wet)


_LOG2E = 1.4426950408889634
_LN2 = 0.6931471805599453


def _p2_body(x_ref, s1_ref, s2_ref, g1_ref, b1_ref, ns_ref, t1_ref, t2_ref):
    i = pl.program_id(0)
    inv = 1.0 / float(E)
    mu = s1_ref[...] * inv
    var = s2_ref[...] * inv - mu * mu
    a1 = g1_ref[...] * lax.rsqrt(var + EPS)
    c1 = b1_ref[...] - mu * a1
    af = -(a1[:, :ATOM] * _LOG2E)
    cf = -(c1[:, :ATOM] * _LOG2E)
    ac = a1[:, ATOM:] * _LOG2E
    cc = c1[:, ATOM:] * _LOG2E
    xf = x_ref[:, :, :ATOM].astype(jnp.float32)
    xc = x_ref[:, :, ATOM:].astype(jnp.float32)
    filt = 1.0 / (1.0 + jnp.exp2(af[None] * xf + cf[None]))
    core = jnp.log2(1.0 + jnp.exp2(ac[None] * xc + cc[None]))
    ns = _LN2 * jnp.sum(filt * core, axis=0)
    ns_ref[...] = ns

    @pl.when(i == 0)
    def _():
        t1_ref[...] = jnp.zeros_like(t1_ref)
        t2_ref[...] = jnp.zeros_like(t2_ref)

    rem = N - i * _TN1
    rmask = lax.broadcasted_iota(jnp.int32, (_TN1, ATOM), 0) < rem
    nsv = jnp.where(rmask, ns, 0.0)
    t1_ref[...] += jnp.sum(nsv, axis=0).reshape(1, ATOM)
    t2_ref[...] += jnp.sum(nsv * nsv, axis=0).reshape(1, ATOM)


def _pass2(x, s1, s2, g1, b1):
    return pl.pallas_call(
        _p2_body,
        grid=(_G1,),
        in_specs=[
            pl.BlockSpec((M, _TN1, F), lambda i: (0, i, 0)),
            pl.BlockSpec((1, F), lambda i: (0, 0)),
            pl.BlockSpec((1, F), lambda i: (0, 0)),
            pl.BlockSpec((1, F), lambda i: (0, 0)),
            pl.BlockSpec((1, F), lambda i: (0, 0)),
        ],
        out_specs=[
            pl.BlockSpec((_TN1, ATOM), lambda i: (i, 0)),
            pl.BlockSpec((1, ATOM), lambda i: (0, 0)),
            pl.BlockSpec((1, ATOM), lambda i: (0, 0)),
        ],
        out_shape=[
            jax.ShapeDtypeStruct((N, ATOM), jnp.float32),
            jax.ShapeDtypeStruct((1, ATOM), jnp.float32),
            jax.ShapeDtypeStruct((1, ATOM), jnp.float32),
        ],
    )(x, s1, s2, g1, b1)


_TN3 = 2000
_G3 = N // _TN3


def _p3_body(atom_ref, ns_ref, t1_ref, t2_ref, g2_ref, b2_ref, out_ref):
    inv = 1.0 / float(N)
    mu2 = t1_ref[...] * inv
    var2 = t2_ref[...] * inv - mu2 * mu2
    a2c = g2_ref[...] * lax.rsqrt(var2 + EPS)
    c2c = b2_ref[...] - mu2 * a2c
    out_ref[...] = jax.nn.softplus(atom_ref[...] + a2c * ns_ref[...] + c2c)


def _pass3(atom, ns, t1, t2, g2, b2):
    return pl.pallas_call(
        _p3_body,
        grid=(_G3,),
        in_specs=[
            pl.BlockSpec((_TN3, ATOM), lambda i: (i, 0)),
            pl.BlockSpec((_TN3, ATOM), lambda i: (i, 0)),
            pl.BlockSpec((1, ATOM), lambda i: (0, 0)),
            pl.BlockSpec((1, ATOM), lambda i: (0, 0)),
            pl.BlockSpec((1, ATOM), lambda i: (0, 0)),
            pl.BlockSpec((1, ATOM), lambda i: (0, 0)),
        ],
        out_specs=pl.BlockSpec((_TN3, ATOM), lambda i: (i, 0)),
        out_shape=jax.ShapeDtypeStruct((N, ATOM), jnp.float32),
    )(atom, ns, t1, t2, g2, b2)


def kernel(atom_in_fea, nbr_fea, nbr_fea_idx, W, b, bn1_g, bn1_b, bn2_g,
           bn2_b):
    atomt = atom_in_fea.T
    nbrt = jnp.transpose(nbr_fea, (2, 1, 0))
    idx_t = nbr_fea_idx.T.astype(jnp.int32)
    wst = W[:, :ATOM].T
    wnt = W[:, ATOM:2 * ATOM].T
    wet = W[:, 2 * ATOM:].T
    b2d = b.reshape(1, F)

    p_tbl, s_tbl = _k0(atomt, wst, wnt, b2d)
    g = _sc_gather(p_tbl, idx_t)
    g3 = g.reshape(M, N, F)
    x, s1, s2 = _pass1(nbrt, g3, s_tbl, wet)
    ns, t1, t2 = _pass2(x, s1, s2, bn1_g.reshape(1, F), bn1_b.reshape(1, F))
    return _pass3(atom_in_fea, ns, t1, t2, bn2_g.reshape(1, ATOM),
                  bn2_b.reshape(1, ATOM))


# speedup vs baseline: 1.3187x; 1.0141x over previous
import functools

import jax
import jax.numpy as jnp
from jax import lax
from jax.experimental import pallas as pl
from jax.experimental.pallas import tpu as pltpu
from jax.experimental.pallas import tpu_sc as plsc

N = 50000
M = 16
ATOM = 64
NBR = 41
F = 2 * ATOM
E = N * M
EPS = 1e-5


_TN0 = 2048
_G0 = (N + _TN0 - 1) // _TN0


def _k0_body(atomt_ref, wst_ref, wnt_ref, b_ref, p_ref, s_ref):
    at = atomt_ref[...]
    dn = (((0,), (0,)), ((), ()))
    p_ref[...] = lax.dot_general(at, wnt_ref[...], dn,
                                 preferred_element_type=jnp.float32)
    s_ref[...] = lax.dot_general(at, wst_ref[...], dn,
                                 preferred_element_type=jnp.float32) + b_ref[...]


def _k0(atomt, wst, wnt, b2d):
    return pl.pallas_call(
        _k0_body,
        grid=(_G0,),
        in_specs=[
            pl.BlockSpec((ATOM, _TN0), lambda i: (0, i)),
            pl.BlockSpec((ATOM, F), lambda i: (0, 0)),
            pl.BlockSpec((ATOM, F), lambda i: (0, 0)),
            pl.BlockSpec((1, F), lambda i: (0, 0)),
        ],
        out_specs=[
            pl.BlockSpec((_TN0, F), lambda i: (i, 0)),
            pl.BlockSpec((_TN0, F), lambda i: (i, 0)),
        ],
        out_shape=[
            jax.ShapeDtypeStruct((N, F), jnp.float32),
            jax.ShapeDtypeStruct((N, F), jnp.float32),
        ],
    )(atomt, wst, wnt, b2d)


_NC, _NS = 2, 16
_NW = _NC * _NS
_HN = N // 2
_CH = 200
_NCH = _HN // _CH


def _sc_gather(p_tbl, idx_t):
    mesh = plsc.VectorSubcoreMesh(core_axis_name="c", subcore_axis_name="s",
                                  num_cores=_NC)

    @functools.partial(
        pl.kernel,
        mesh=mesh,
        out_type=jax.ShapeDtypeStruct((E, F), jnp.float32),
        scratch_types=[
            pltpu.VMEM((N,), jnp.int32),
            pltpu.VMEM((_CH, F), jnp.float32),
            pltpu.VMEM((_CH, F), jnp.float32),
            pltpu.SemaphoreType.DMA,
            pltpu.SemaphoreType.DMA,
            pltpu.SemaphoreType.DMA,
            pltpu.SemaphoreType.DMA,
        ],
    )
    def gather_k(tbl_hbm, idx_hbm, out_hbm, idx_v, rows0, rows1, sem_g0,
                 sem_g1, sem_w0, sem_w1):
        wid = lax.axis_index("s") * _NC + lax.axis_index("c")
        m = wid // 2
        base = lax.rem(wid, 2) * _HN
        pltpu.sync_copy(idx_hbm.at[m], idx_v)
        obase = m * N + base

        def issue_gather(ch, rows_v, sem_g):
            pltpu.async_copy(
                tbl_hbm.at[idx_v.at[pl.ds(base + ch * _CH, _CH)]], rows_v,
                sem_g)

        issue_gather(0, rows0, sem_g0)

        def body(ch, _):
            b = lax.rem(ch, 2)

            def chunk(rows_v, sem_g, sem_w, rows_n, sem_gn, sem_wn):
                @pl.when(ch + 1 < _NCH)
                def _():
                    @pl.when(ch >= 1)
                    def _():
                        pltpu.make_async_copy(
                            rows_n, out_hbm.at[pl.ds(obase, _CH)],
                            sem_wn).wait()

                    issue_gather(ch + 1, rows_n, sem_gn)

                pltpu.make_async_copy(
                    tbl_hbm.at[idx_v.at[pl.ds(base, _CH)]], rows_v,
                    sem_g).wait()
                pltpu.async_copy(rows_v, out_hbm.at[pl.ds(obase + ch * _CH,
                                                          _CH)], sem_w)

            @pl.when(b == 0)
            def _():
                chunk(rows0, sem_g0, sem_w0, rows1, sem_g1, sem_w1)

            @pl.when(b == 1)
            def _():
                chunk(rows1, sem_g1, sem_w1, rows0, sem_g0, sem_w0)

            return 0

        lax.fori_loop(0, _NCH, body, 0)
        pltpu.make_async_copy(rows0, out_hbm.at[pl.ds(obase, _CH)],
                              sem_w0).wait()
        pltpu.make_async_copy(rows1, out_hbm.at[pl.ds(obase, _CH)],
                              sem_w1).wait()

    return gather_k(p_tbl, idx_t)


_TN1 = 1024
_G1 = (N + _TN1 - 1) // _TN1


def _p1_body(nbrt_ref, g_ref, s_ref, wet_ref, x_ref, s1_ref, s2_ref):
    i = pl.program_id(0)
    s = s_ref[...]
    rem = N - i * _TN1
    rmask = lax.broadcasted_iota(jnp.int32, (_TN1, F), 0) < rem
    dn = (((0,), (0,)), ((), ()))
    acc1 = jnp.zeros((1, F), jnp.float32)
    acc2 = jnp.zeros((1, F), jnp.float32)
    for m in range(M):
        ey = lax.dot_general(nbrt_ref[:, m, :], wet_ref[...], dn,
                             preferred_element_type=jnp.float32)
        xm = g_ref[m] + ey + s
        x_ref[m] = xm.astype(jnp.bfloat16)
        xv = jnp.where(rmask, xm, 0.0)
        acc1 += jnp.sum(xv, axis=0).reshape(1, F)
        acc2 += jnp.sum(xv * xv, axis=0).reshape(1, F)

    @pl.when(i == 0)
    def _():
        s1_ref[...] = jnp.zeros_like(s1_ref)
        s2_ref[...] = jnp.zeros_like(s2_ref)

    s1_ref[...] += acc1
    s2_ref[...] += acc2


def _pass1(nbrt, g3, s, wet):
    return pl.pallas_call(
        _p1_body,
        grid=(_G1,),
        in_specs=[
            pl.BlockSpec((NBR, M, _TN1), lambda i: (0, 0, i)),
            pl.BlockSpec((M, _TN1, F), lambda i: (0, i, 0)),
            pl.BlockSpec((_TN1, F), lambda i: (i, 0)),
            pl.BlockSpec((NBR, F), lambda i: (0, 0)),
        ],
        out_specs=[
            pl.BlockSpec((M, _TN1, F), lambda i: (0, i, 0)),
            pl.BlockSpec((1, F), lambda i: (0, 0)),
            pl.BlockSpec((1, F), lambda i: (0, 0)),
        ],
        out_shape=[
            jax.ShapeDtypeStruct((M, N, F), jnp.bfloat16),
            jax.ShapeDtypeStruct((1, F), jnp.float32),
            jax.ShapeDtypeStruct((1, F), jnp.float32),
        ],
    )(nbrt, g3, s, wet)


_LOG2E = 1.4426950408889634
_LN2 = 0.6931471805599453


def _p2_body(x_ref, s1_ref, s2_ref, g1_ref, b1_ref, ns_ref, t1_ref, t2_ref):
    i = pl.program_id(0)
    inv = 1.0 / float(E)
    mu = s1_ref[...] * inv
    var = s2_ref[...] * inv - mu * mu
    a1 = g1_ref[...] * lax.rsqrt(var + EPS)
    c1 = b1_ref[...] - mu * a1
    af = -(a1[:, :ATOM] * _LOG2E)
    cf = -(c1[:, :ATOM] * _LOG2E)
    ac = a1[:, ATOM:] * _LOG2E
    cc = c1[:, ATOM:] * _LOG2E
    xf = x_ref[:, :, :ATOM].astype(jnp.float32)
    xc = x_ref[:, :, ATOM:].astype(jnp.float32)
    filt = 1.0 / (1.0 + jnp.exp2(af[None] * xf + cf[None]))
    core = jnp.log2(1.0 + jnp.exp2(ac[None] * xc + cc[None]))
    ns = _LN2 * jnp.sum(filt * core, axis=0)
    ns_ref[...] = ns

    @pl.when(i == 0)
    def _():
        t1_ref[...] = jnp.zeros_like(t1_ref)
        t2_ref[...] = jnp.zeros_like(t2_ref)

    rem = N - i * _TN1
    rmask = lax.broadcasted_iota(jnp.int32, (_TN1, ATOM), 0) < rem
    nsv = jnp.where(rmask, ns, 0.0)
    t1_ref[...] += jnp.sum(nsv, axis=0).reshape(1, ATOM)
    t2_ref[...] += jnp.sum(nsv * nsv, axis=0).reshape(1, ATOM)


def _pass2(x, s1, s2, g1, b1):
    return pl.pallas_call(
        _p2_body,
        grid=(_G1,),
        in_specs=[
            pl.BlockSpec((M, _TN1, F), lambda i: (0, i, 0)),
            pl.BlockSpec((1, F), lambda i: (0, 0)),
            pl.BlockSpec((1, F), lambda i: (0, 0)),
            pl.BlockSpec((1, F), lambda i: (0, 0)),
            pl.BlockSpec((1, F), lambda i: (0, 0)),
        ],
        out_specs=[
            pl.BlockSpec((_TN1, ATOM), lambda i: (i, 0)),
            pl.BlockSpec((1, ATOM), lambda i: (0, 0)),
            pl.BlockSpec((1, ATOM), lambda i: (0, 0)),
        ],
        out_shape=[
            jax.ShapeDtypeStruct((N, ATOM), jnp.float32),
            jax.ShapeDtypeStruct((1, ATOM), jnp.float32),
            jax.ShapeDtypeStruct((1, ATOM), jnp.float32),
        ],
    )(x, s1, s2, g1, b1)


_TN3 = 2000
_G3 = N // _TN3


def _p3_body(atom_ref, ns_ref, t1_ref, t2_ref, g2_ref, b2_ref, out_ref):
    inv = 1.0 / float(N)
    mu2 = t1_ref[...] * inv
    var2 = t2_ref[...] * inv - mu2 * mu2
    a2c = g2_ref[...] * lax.rsqrt(var2 + EPS)
    c2c = b2_ref[...] - mu2 * a2c
    out_ref[...] = jax.nn.softplus(atom_ref[...] + a2c * ns_ref[...] + c2c)


def _pass3(atom, ns, t1, t2, g2, b2):
    return pl.pallas_call(
        _p3_body,
        grid=(_G3,),
        in_specs=[
            pl.BlockSpec((_TN3, ATOM), lambda i: (i, 0)),
            pl.BlockSpec((_TN3, ATOM), lambda i: (i, 0)),
            pl.BlockSpec((1, ATOM), lambda i: (0, 0)),
            pl.BlockSpec((1, ATOM), lambda i: (0, 0)),
            pl.BlockSpec((1, ATOM), lambda i: (0, 0)),
            pl.BlockSpec((1, ATOM), lambda i: (0, 0)),
        ],
        out_specs=pl.BlockSpec((_TN3, ATOM), lambda i: (i, 0)),
        out_shape=jax.ShapeDtypeStruct((N, ATOM), jnp.float32),
    )(atom, ns, t1, t2, g2, b2)


def kernel(atom_in_fea, nbr_fea, nbr_fea_idx, W, b, bn1_g, bn1_b, bn2_g,
           bn2_b):
    atomt = atom_in_fea.T
    nbrt = jnp.transpose(nbr_fea, (2, 1, 0))
    idx_t = nbr_fea_idx.T.astype(jnp.int32)
    wst = W[:, :ATOM].T
    wnt = W[:, ATOM:2 * ATOM].T
    wet = W[:, 2 * ATOM:].T
    b2d = b.reshape(1, F)

    p_tbl, s_tbl = _k0(atomt, wst, wnt, b2d)
    g = _sc_gather(p_tbl, idx_t)
    g3 = g.reshape(M, N, F)
    x, s1, s2 = _pass1(nbrt, g3, s_tbl, wet)
    ns, t1, t2 = _pass2(x, s1, s2, bn1_g.reshape(1, F), bn1_b.reshape(1, F))
    return _pass3(atom_in_fea, ns, t1, t2, bn2_g.reshape(1, ATOM),
                  bn2_b.reshape(1, ATOM))
